# Initial kernel scaffold; baseline (speedup 1.0000x reference)
import jax
import jax.numpy as jnp
from jax.experimental import pallas as pl


def kernel(edge_attr, W0, W1, W2):
    raise NotImplementedError("write your pallas kernel here")


# speedup vs baseline: 7.5414x; 7.5414x over previous
import functools

import jax
import jax.numpy as jnp
from jax import lax
from jax.experimental import pallas as pl
from jax.experimental.pallas import tpu as pltpu
from jax.experimental.pallas import tpu_sc as plsc

EMB = 128
D0, D1, D2 = 6, 7, 3
NROWS = D0 * D1 * D2
CHUNK = 128
NW = 32


def _encoder_call(E):
    nchunks = E // CHUNK
    iters = (nchunks + NW - 1) // NW
    mesh = plsc.VectorSubcoreMesh(core_axis_name="c", subcore_axis_name="s")

    @functools.partial(
        pl.kernel,
        out_type=jax.ShapeDtypeStruct((E, EMB), jnp.float32),
        mesh=mesh,
        scratch_types=[
            pltpu.VMEM((D0, EMB), jnp.float32),
            pltpu.VMEM((D1, EMB), jnp.float32),
            pltpu.VMEM((D2, EMB), jnp.float32),
            pltpu.VMEM((NROWS, EMB), jnp.float32),
            pltpu.VMEM_SHARED((NROWS, EMB), jnp.float32),
            pltpu.VMEM((CHUNK,), jnp.int32),
            pltpu.VMEM((CHUNK,), jnp.int32),
            pltpu.VMEM((CHUNK,), jnp.int32),
            pltpu.VMEM((CHUNK,), jnp.int32),
            pltpu.VMEM((CHUNK, EMB), jnp.float32),
            pltpu.SemaphoreType.DMA,
        ],
    )
    def k(a0, a1, a2, w0, w1, w2, out, w0_v, w1_v, w2_v, t_v, t_sh,
          i0, i1, i2, cb, rows, sem):
        cid = lax.axis_index("c")
        sid = lax.axis_index("s")
        wid = sid * 2 + cid

        @pl.when(sid == 0)
        def _build_table():
            pltpu.sync_copy(w0, w0_v)
            pltpu.sync_copy(w1, w1_v)
            pltpu.sync_copy(w2, w2_v)

            def row(r, _):
                r0 = r // (D1 * D2)
                rem = r % (D1 * D2)
                r1 = rem // D2
                r2 = rem % D2

                def seg(si, _):
                    o = si * 16
                    t_v[r, pl.ds(o, 16)] = (
                        w0_v[r0, pl.ds(o, 16)]
                        + w1_v[r1, pl.ds(o, 16)]
                        + w2_v[r2, pl.ds(o, 16)]
                    )
                    return _

                lax.fori_loop(0, EMB // 16, seg, None)
                return _

            lax.fori_loop(0, NROWS, row, None)
            pltpu.sync_copy(t_v, t_sh)

        plsc.subcore_barrier()

        def chunk(j, _):
            t = j * NW + wid

            @pl.when(t < nchunks)
            def _do():
                base = t * CHUNK
                pltpu.sync_copy(a0.at[pl.ds(base, CHUNK)], i0)
                pltpu.sync_copy(a1.at[pl.ds(base, CHUNK)], i1)
                pltpu.sync_copy(a2.at[pl.ds(base, CHUNK)], i2)

                def cgen(i, _):
                    o = i * 16
                    v0 = jnp.minimum(i0[pl.ds(o, 16)], D0 - 1)
                    v1 = jnp.minimum(i1[pl.ds(o, 16)], D1 - 1)
                    v2 = jnp.minimum(i2[pl.ds(o, 16)], D2 - 1)
                    cb[pl.ds(o, 16)] = v0 * (D1 * D2) + v1 * D2 + v2
                    return _

                lax.fori_loop(0, CHUNK // 16, cgen, None)
                pltpu.async_copy(t_sh.at[cb], rows, sem).wait()
                pltpu.sync_copy(rows, out.at[pl.ds(base, CHUNK)])

            return _

        lax.fori_loop(0, iters, chunk, None)

    return k


def kernel(edge_attr, W0, W1, W2):
    E = edge_attr.shape[0]
    idx = edge_attr.astype(jnp.int32)
    a0 = idx[:, 0]
    a1 = idx[:, 1]
    a2 = idx[:, 2]
    return _encoder_call(E)(a0, a1, a2, W0, W1, W2)


# speedup vs baseline: 17.2063x; 2.2816x over previous
import functools

import jax
import jax.numpy as jnp
from jax import lax
from jax.experimental import pallas as pl
from jax.experimental.pallas import tpu as pltpu
from jax.experimental.pallas import tpu_sc as plsc

EMB = 128
D0, D1, D2 = 6, 7, 3
NROWS = D0 * D1 * D2
CHUNK = 128
NW = 32


def _encoder_call(E):
    nchunks = E // CHUNK
    full_rounds = nchunks // NW
    tail = nchunks - full_rounds * NW
    pairs = full_rounds // 2
    odd_round = full_rounds - pairs * 2
    mesh = plsc.VectorSubcoreMesh(core_axis_name="c", subcore_axis_name="s")

    @functools.partial(
        pl.kernel,
        out_type=jax.ShapeDtypeStruct((E, EMB), jnp.float32),
        mesh=mesh,
        scratch_types=[
            pltpu.VMEM((D0, EMB), jnp.float32),
            pltpu.VMEM((D1, EMB), jnp.float32),
            pltpu.VMEM((D2, EMB), jnp.float32),
            pltpu.VMEM((NROWS, EMB), jnp.float32),
            pltpu.VMEM_SHARED((NROWS, EMB), jnp.float32),
            pltpu.VMEM((2, 3, CHUNK), jnp.int32),
            pltpu.VMEM((2, CHUNK), jnp.int32),
            pltpu.VMEM((CHUNK, EMB), jnp.float32),
            pltpu.VMEM((CHUNK, EMB), jnp.float32),
            pltpu.SemaphoreType.DMA,
            pltpu.SemaphoreType.DMA,
            pltpu.SemaphoreType.DMA,
            pltpu.SemaphoreType.DMA,
        ],
    )
    def k(a0, a1, a2, w0, w1, w2, out, w0_v, w1_v, w2_v, t_v, t_sh,
          i3, cb2, rows0, rows1, isem, gsem, wsem0, wsem1):
        cid = lax.axis_index("c")
        sid = lax.axis_index("s")
        wid = sid * 2 + cid
        rows = (rows0, rows1)
        wsem = (wsem0, wsem1)
        srcs = (a0, a1, a2)

        def fetch_idx(b, t):
            base = t * CHUNK
            for ki in range(3):
                pltpu.async_copy(
                    srcs[ki].at[pl.ds(base, CHUNK)], i3.at[b, ki], isem)

        def wait_idx(b):
            for ki in range(3):
                pltpu.make_async_copy(
                    srcs[ki].at[pl.ds(0, CHUNK)], i3.at[b, ki], isem).wait()

        def compute(b):
            def cgen(i, carry):
                o = i * 16
                v0 = jnp.minimum(i3[b, 0, pl.ds(o, 16)], D0 - 1)
                v1 = jnp.minimum(i3[b, 1, pl.ds(o, 16)], D1 - 1)
                v2 = jnp.minimum(i3[b, 2, pl.ds(o, 16)], D2 - 1)
                cb2[b, pl.ds(o, 16)] = v0 * (D1 * D2) + v1 * D2 + v2
                return carry

            lax.fori_loop(0, CHUNK // 16, cgen, None)

        def gather(b):
            pltpu.async_copy(t_sh.at[cb2.at[b]], rows[b], gsem).wait()

        def write(b, t):
            pltpu.async_copy(
                rows[b], out.at[pl.ds(t * CHUNK, CHUNK)], wsem[b])

        def wait_write(b):
            pltpu.make_async_copy(
                rows[b], out.at[pl.ds(0, CHUNK)], wsem[b]).wait()

        fetch_idx(0, wid)

        @pl.when(sid == 0)
        def _build_table():
            pltpu.sync_copy(w0, w0_v)
            pltpu.sync_copy(w1, w1_v)
            pltpu.sync_copy(w2, w2_v)

            def row(r, carry):
                r0 = r // (D1 * D2)
                rem = r % (D1 * D2)
                r1 = rem // D2
                r2 = rem % D2

                def seg(si, c2):
                    o = si * 16
                    t_v[r, pl.ds(o, 16)] = (
                        w0_v[r0, pl.ds(o, 16)]
                        + w1_v[r1, pl.ds(o, 16)]
                        + w2_v[r2, pl.ds(o, 16)]
                    )
                    return c2

                lax.fori_loop(0, EMB // 16, seg, None)
                return carry

            lax.fori_loop(0, NROWS, row, None)
            pltpu.sync_copy(t_v, t_sh)

        plsc.subcore_barrier()

        def do_round(jj, carry):
            for b in range(2):
                j = jj * 2 + b
                t = j * NW + wid
                wait_idx(b)
                compute(b)

                @pl.when(j < full_rounds - 1)
                def _prefetch():
                    fetch_idx(1 - b, (j + 1) * NW + wid)

                @pl.when(jj >= 1)
                def _drain():
                    wait_write(b)

                gather(b)
                write(b, t)
            return carry

        lax.fori_loop(0, pairs, do_round, None)

        if odd_round:
            j = pairs * 2
            t = j * NW + wid
            wait_idx(0)
            compute(0)
            if pairs >= 1:
                wait_write(0)
            gather(0)
            write(0, t)

        if full_rounds >= 2:
            wait_write(0)
            wait_write(1)
        elif full_rounds == 1:
            wait_write(0)

        if tail:
            @pl.when(wid < tail)
            def _tail():
                t = full_rounds * NW + wid
                fetch_idx(0, t)
                wait_idx(0)
                compute(0)
                gather(0)
                write(0, t)
                wait_write(0)

    return k


def kernel(edge_attr, W0, W1, W2):
    E = edge_attr.shape[0]
    idx = edge_attr.astype(jnp.int32)
    a0 = idx[:, 0]
    a1 = idx[:, 1]
    a2 = idx[:, 2]
    return _encoder_call(E)(a0, a1, a2, W0, W1, W2)


# speedup vs baseline: 17.7078x; 1.0291x over previous
import functools

import jax
import jax.numpy as jnp
from jax import lax
from jax.experimental import pallas as pl
from jax.experimental.pallas import tpu as pltpu
from jax.experimental.pallas import tpu_sc as plsc

EMB = 128
D0, D1, D2 = 6, 7, 3
NROWS = D0 * D1 * D2
CHUNK = 128
NIDX = 128
NW = 32


def _encoder_call(E):
    nchunks = E // CHUNK
    full_rounds = nchunks // NW
    tail = nchunks - full_rounds * NW
    pairs = full_rounds // 2
    odd_round = full_rounds - pairs * 2
    mesh = plsc.VectorSubcoreMesh(core_axis_name="c", subcore_axis_name="s")

    @functools.partial(
        pl.kernel,
        out_type=jax.ShapeDtypeStruct((E, EMB), jnp.float32),
        mesh=mesh,
        scratch_types=[
            pltpu.VMEM((D0, EMB), jnp.float32),
            pltpu.VMEM((D1, EMB), jnp.float32),
            pltpu.VMEM((D2, EMB), jnp.float32),
            pltpu.VMEM((NROWS, EMB), jnp.float32),
            pltpu.VMEM_SHARED((NROWS, EMB), jnp.float32),
            pltpu.VMEM((2, 3, CHUNK), jnp.int32),
            pltpu.VMEM((2, CHUNK), jnp.int32),
            pltpu.VMEM((CHUNK, EMB), jnp.float32),
            pltpu.VMEM((CHUNK, EMB), jnp.float32),
            pltpu.SemaphoreType.DMA,
            pltpu.SemaphoreType.DMA,
            pltpu.SemaphoreType.DMA,
            pltpu.SemaphoreType.DMA,
            pltpu.SemaphoreType.DMA,
        ],
    )
    def k(a0, a1, a2, w0, w1, w2, out, w0_v, w1_v, w2_v, t_v, t_sh,
          i3, cb2, rows0, rows1, isem, gsem0, gsem1, wsem0, wsem1):
        cid = lax.axis_index("c")
        sid = lax.axis_index("s")
        wid = sid * 2 + cid
        rows = (rows0, rows1)
        gsem = (gsem0, gsem1)
        wsem = (wsem0, wsem1)
        srcs = (a0, a1, a2)

        def fetch_idx(b, t):
            base = t * CHUNK
            for ki in range(3):
                pltpu.async_copy(
                    srcs[ki].at[pl.ds(base, CHUNK)], i3.at[b, ki], isem)

        def wait_idx(b):
            for ki in range(3):
                pltpu.make_async_copy(
                    srcs[ki].at[pl.ds(0, CHUNK)], i3.at[b, ki], isem).wait()

        def compute(b):
            def cgen(i, carry):
                o = i * 16
                v0 = jnp.minimum(i3[b, 0, pl.ds(o, 16)], D0 - 1)
                v1 = jnp.minimum(i3[b, 1, pl.ds(o, 16)], D1 - 1)
                v2 = jnp.minimum(i3[b, 2, pl.ds(o, 16)], D2 - 1)
                cb2[b, pl.ds(o, 16)] = v0 * (D1 * D2) + v1 * D2 + v2
                return carry

            lax.fori_loop(0, CHUNK // 16, cgen, None)

        def gather_start(b):
            for kk in range(CHUNK // NIDX):
                pltpu.async_copy(
                    t_sh.at[cb2.at[b, pl.ds(kk * NIDX, NIDX)]],
                    rows[b].at[pl.ds(kk * NIDX, NIDX)],
                    gsem[b])

        def gather_wait(b):
            for kk in range(CHUNK // NIDX):
                pltpu.make_async_copy(
                    t_sh.at[cb2.at[b, pl.ds(kk * NIDX, NIDX)]],
                    rows[b].at[pl.ds(kk * NIDX, NIDX)],
                    gsem[b]).wait()

        def write(b, t):
            pltpu.async_copy(
                rows[b], out.at[pl.ds(t * CHUNK, CHUNK)], wsem[b])

        def wait_write(b):
            pltpu.make_async_copy(
                rows[b], out.at[pl.ds(0, CHUNK)], wsem[b]).wait()

        fetch_idx(0, wid)

        @pl.when(sid == 0)
        def _build_table():
            pltpu.sync_copy(w0, w0_v)
            pltpu.sync_copy(w1, w1_v)
            pltpu.sync_copy(w2, w2_v)

            def row(r, carry):
                r0 = r // (D1 * D2)
                rem = r % (D1 * D2)
                r1 = rem // D2
                r2 = rem % D2

                def seg(si, c2):
                    o = si * 16
                    t_v[r, pl.ds(o, 16)] = (
                        w0_v[r0, pl.ds(o, 16)]
                        + w1_v[r1, pl.ds(o, 16)]
                        + w2_v[r2, pl.ds(o, 16)]
                    )
                    return c2

                lax.fori_loop(0, EMB // 16, seg, None)
                return carry

            lax.fori_loop(0, NROWS, row, None)
            pltpu.sync_copy(t_v, t_sh)

        plsc.subcore_barrier()

        def do_round(jj, carry):
            for b in range(2):
                j = jj * 2 + b
                t = j * NW + wid
                wait_idx(b)
                compute(b)

                @pl.when(j < full_rounds - 1)
                def _prefetch():
                    fetch_idx(1 - b, (j + 1) * NW + wid)

                @pl.when(jj >= 1)
                def _recycle():
                    wait_write(b)

                gather_start(b)

                if b == 1:
                    gather_wait(0)
                    write(0, t - NW)
                else:
                    @pl.when(jj >= 1)
                    def _flush_prev():
                        gather_wait(1)
                        write(1, t - NW)
            return carry

        lax.fori_loop(0, pairs, do_round, None)

        if odd_round:
            j = pairs * 2
            t = j * NW + wid
            wait_idx(0)
            compute(0)
            if pairs >= 1:
                wait_write(0)
            gather_start(0)
            if full_rounds >= 2:
                gather_wait(1)
                write(1, t - NW)

        if full_rounds >= 1:
            bl = (full_rounds - 1) % 2
            gather_wait(bl)
            write(bl, (full_rounds - 1) * NW + wid)
            wait_write(bl)
            if full_rounds >= 2:
                wait_write(1 - bl)

        if tail:
            @pl.when(wid < tail)
            def _tail():
                t = full_rounds * NW + wid
                fetch_idx(0, t)
                wait_idx(0)
                compute(0)
                gather_start(0)
                gather_wait(0)
                write(0, t)
                wait_write(0)

    return k


def kernel(edge_attr, W0, W1, W2):
    E = edge_attr.shape[0]
    idx = edge_attr.astype(jnp.int32)
    a0 = idx[:, 0]
    a1 = idx[:, 1]
    a2 = idx[:, 2]
    return _encoder_call(E)(a0, a1, a2, W0, W1, W2)
